# baseline (speedup 1.0000x reference)
import functools

import jax
import jax.numpy as jnp
from jax import lax
from jax.experimental import pallas as pl
from jax.experimental.pallas import tpu as pltpu
from jax.experimental.pallas import tpu_sc as plsc

N = 10000
F = 128
FH = F // 2
E = 320000
ER = E // 128
BLK = 1000

NC = 2
NS = 16
ROWS_T = 157
NP = 78
DEG_N = 10240
ORPT = DEG_N // NS


def _spmm_body(srch_hbm, dsth_hbm, w_hbm, xflat_hbm,
               out_tx, out_degp, out_dis,
               src2d, dst2d, w2d, degloc, degtmp, rows,
               tx1_sh, gsem, ssem):
    c = lax.axis_index("c")
    s = lax.axis_index("s")
    xoff = c * N
    off = 156 * s + jnp.minimum(s, 4)

    zero16f = jnp.zeros((16,), jnp.float32)
    zero16i = jnp.zeros((16,), jnp.int32)

    pltpu.sync_copy(srch_hbm.at[pl.ds(off, 156)], src2d.at[pl.ds(0, 156)])
    pltpu.sync_copy(dsth_hbm.at[pl.ds(off, 156)], dst2d.at[pl.ds(0, 156)])
    pltpu.sync_copy(w_hbm.at[pl.ds(off, 156)], w2d.at[pl.ds(0, 156)])

    @pl.when(s < 4)
    def _load_extra_row():
        pltpu.sync_copy(srch_hbm.at[pl.ds(off + 156, 1)],
                        src2d.at[pl.ds(156, 1)])
        pltpu.sync_copy(dsth_hbm.at[pl.ds(off + 156, 1)],
                        dst2d.at[pl.ds(156, 1)])
        pltpu.sync_copy(w_hbm.at[pl.ds(off + 156, 1)], w2d.at[pl.ds(156, 1)])

    @pl.when(s >= 4)
    def _zero_tail_row():
        for k in range(8):
            src2d[156, pl.ds(16 * k, 16)] = zero16i
            dst2d[156, pl.ds(16 * k, 16)] = zero16i
            w2d[156, pl.ds(16 * k, 16)] = zero16f

    def zr(i, _):
        degloc[pl.ds(16 * i, 16)] = zero16f
        return 0
    lax.fori_loop(0, DEG_N // 16, zr, 0)

    def zb(i, _):
        rows[0, lax.div(i, 4), pl.ds(16 * lax.rem(i, 4), 16)] = zero16f
        return 0
    lax.fori_loop(0, 512, zb, 0)

    def zc(k, _):
        pltpu.sync_copy(rows.at[0], tx1_sh.at[pl.ds(ORPT * s + 128 * k, 128)])
        return 0
    lax.fori_loop(0, 5, zc, 0)

    def ar(r, _):
        for j in range(8):
            idx = src2d[r, pl.ds(16 * j, 16)]
            wv = w2d[r, pl.ds(16 * j, 16)]
            plsc.addupdate_scatter(degloc, [idx], wv)
            src2d[r, pl.ds(16 * j, 16)] = idx + xoff
        return 0
    lax.fori_loop(0, ROWS_T, ar, 0)

    pltpu.sync_copy(degloc, out_degp.at[c, s])
    plsc.subcore_barrier()

    for rnd in range(4):
        def ldr(t, _, rnd=rnd):
            pltpu.sync_copy(out_degp.at[c, 4 * rnd + t, pl.ds(ORPT * s, ORPT)],
                            degtmp.at[t])
            return 0
        lax.fori_loop(0, 4, ldr, 0)

        def mr_(i, _, rnd=rnd):
            acc = degtmp[0, pl.ds(16 * i, 16)]
            for t in range(1, 4):
                acc = acc + degtmp[t, pl.ds(16 * i, 16)]
            if rnd > 0:
                acc = acc + degloc[pl.ds(ORPT * s + 16 * i, 16)]
            if rnd == 3:
                iv = plsc.bitcast(acc, jnp.int32)
                y = plsc.bitcast(0x5F3759DF - lax.shift_right_logical(iv, 1),
                                 jnp.float32)
                for _it in range(3):
                    y = y * (1.5 - ((0.5 * acc) * y) * y)
                acc = jnp.where(acc > 0.0, y, 0.0)
            degloc[pl.ds(ORPT * s + 16 * i, 16)] = acc
            return 0
        lax.fori_loop(0, ORPT // 16, mr_, 0)

    pltpu.sync_copy(degloc.at[pl.ds(ORPT * s, ORPT)],
                    out_dis.at[c, pl.ds(ORPT * s, ORPT)])
    plsc.subcore_barrier()
    pltpu.sync_copy(out_dis.at[c], degloc)

    def nr(r, _):
        for j in range(8):
            si = src2d[r, pl.ds(16 * j, 16)]
            di = dst2d[r, pl.ds(16 * j, 16)]
            wv = w2d[r, pl.ds(16 * j, 16)]
            dsv = plsc.load_gather(degloc, [si - xoff])
            ddv = plsc.load_gather(degloc, [di])
            w2d[r, pl.ds(16 * j, 16)] = (-wv) * dsv * ddv
        return 0
    lax.fori_loop(0, ROWS_T, nr, 0)

    def scale_row(buf, r):
        def er(g, _):
            nv = w2d[r, pl.ds(16 * g, 16)]
            for e in range(16):
                nb = jnp.full((16,), nv[e], jnp.float32)
                for k in range(4):
                    rows[buf, 16 * g + e, pl.ds(16 * k, 16)] = (
                        rows[buf, 16 * g + e, pl.ds(16 * k, 16)] * nb)
            return 0
        lax.fori_loop(0, 8, er, 0, unroll=8)

    pltpu.async_copy(xflat_hbm.at[src2d.at[0]], rows.at[0], gsem.at[0])

    def cr(r, _):
        cur = lax.rem(r, 2)
        nxt = 1 - cur

        @pl.when(r >= 1)
        def _wait_prev_scatter():
            pltpu.make_async_copy(
                rows.at[nxt], tx1_sh.at[dst2d.at[r - 1]],
                ssem.at[nxt]).wait()

        @pl.when(r + 1 < ROWS_T)
        def _prefetch_next():
            pltpu.async_copy(xflat_hbm.at[src2d.at[r + 1]],
                             rows.at[nxt], gsem.at[nxt])

        pltpu.make_async_copy(xflat_hbm.at[src2d.at[r]],
                              rows.at[cur], gsem.at[cur]).wait()

        scale_row(cur, r)
        pltpu.async_copy(rows.at[cur], tx1_sh.at[dst2d.at[r]],
                         ssem.at[cur], add=True)
        return 0

    lax.fori_loop(0, ROWS_T, cr, 0)
    pltpu.make_async_copy(rows.at[(ROWS_T - 1) % 2],
                          tx1_sh.at[dst2d.at[ROWS_T - 1]],
                          ssem.at[(ROWS_T - 1) % 2]).wait()

    plsc.subcore_barrier()

    pltpu.sync_copy(tx1_sh.at[pl.ds(ORPT * s, ORPT)],
                    out_tx.at[c, pl.ds(ORPT * s, ORPT)])


@jax.jit
def _spmm_sc(src2, dst2, w2, xflat):
    mesh = plsc.VectorSubcoreMesh(core_axis_name="c", subcore_axis_name="s",
                                  num_cores=NC, num_subcores=NS)
    return pl.kernel(
        _spmm_body,
        out_type=(
            jax.ShapeDtypeStruct((NC, DEG_N, FH), jnp.float32),
            jax.ShapeDtypeStruct((NC, NS, DEG_N), jnp.float32),
            jax.ShapeDtypeStruct((NC, DEG_N), jnp.float32),
        ),
        mesh=mesh,
        compiler_params=pltpu.CompilerParams(needs_layout_passes=False,
                                             use_tc_tiling_on_sc=False),
        scratch_types=[
            pltpu.VMEM((ROWS_T, 128), jnp.int32),
            pltpu.VMEM((ROWS_T, 128), jnp.int32),
            pltpu.VMEM((ROWS_T, 128), jnp.float32),
            pltpu.VMEM((DEG_N,), jnp.float32),
            pltpu.VMEM((4, ORPT), jnp.float32),
            pltpu.VMEM((2, 128, FH), jnp.float32),
            pltpu.VMEM_SHARED((DEG_N, FH), jnp.float32),
            pltpu.SemaphoreType.DMA((2,)),
            pltpu.SemaphoreType.DMA((2,)),
        ],
    )(src2, dst2, w2, xflat)


def _dense_body(x_ref, t0_ref, t1_ref, wz0_ref, wz1a_ref, wz1b_ref,
                wh0_ref, wh1a_ref, wh1b_ref,
                bz_ref, bh_ref, lw_ref, lb_ref, out_ref):
    xb = x_ref[...]
    t0 = t0_ref[0]
    t1 = t1_ref[0]
    z = jax.nn.sigmoid(xb @ wz0_ref[...] + t0 @ wz1a_ref[...]
                       + t1 @ wz1b_ref[...] + bz_ref[...])
    ht = jnp.tanh(xb @ wh0_ref[...] + t0 @ wh1a_ref[...]
                  + t1 @ wh1b_ref[...] + bh_ref[...])
    h = jax.nn.relu((1.0 - z) * ht)
    out_ref[...] = h @ lw_ref[...] + lb_ref[...]


def _dense_block(x, parts, wz0, wz1, wh0, wh1, bz, bh, lw, lb):
    row_spec = pl.BlockSpec((BLK, F), lambda i: (i, 0))
    part0 = pl.BlockSpec((1, BLK, FH), lambda i: (0, i, 0))
    part1 = pl.BlockSpec((1, BLK, FH), lambda i: (1, i, 0))
    full = lambda shape: pl.BlockSpec(shape, lambda i: (0, 0))
    return pl.pallas_call(
        _dense_body,
        grid=(N // BLK,),
        in_specs=[row_spec, part0, part1,
                  full((F, F)), full((FH, F)), full((FH, F)),
                  full((F, F)), full((FH, F)), full((FH, F)),
                  full((1, F)), full((1, F)), full((F, 1)), full((1, 1))],
        out_specs=pl.BlockSpec((BLK, 1), lambda i: (i, 0)),
        out_shape=jax.ShapeDtypeStruct((N, 1), jnp.float32),
    )(x, parts, parts, wz0, wz1[:FH], wz1[FH:], wh0, wh1[:FH], wh1[FH:],
      bz, bh, lw, lb)


def kernel(x, edge_index, edge_weight,
           xz_W0, xz_W1, xz_b, hz_W0, hz_W1, hz_b,
           xr_W0, xr_W1, xr_b, hr_W0, hr_W1, hr_b,
           xh_W0, xh_W1, xh_b, hh_W0, hh_W1, hh_b,
           lin_W, lin_b):
    ei3 = edge_index.reshape(2, ER, 128)
    w2 = edge_weight.reshape(ER, 128)
    xflat = jnp.concatenate([x[:, :FH], x[:, FH:]], axis=0)

    parts, _, _ = _spmm_sc(ei3[0], ei3[1], w2, xflat)

    bz = (xz_b + hz_b).reshape(1, F)
    bh = (xh_b + hh_b).reshape(1, F)
    return _dense_block(x, parts, xz_W0, xz_W1, xh_W0, xh_W1,
                        bz, bh, lin_W, lin_b.reshape(1, 1))


# speedup vs baseline: 1.0567x; 1.0567x over previous
import functools

import jax
import jax.numpy as jnp
from jax import lax
from jax.experimental import pallas as pl
from jax.experimental.pallas import tpu as pltpu
from jax.experimental.pallas import tpu_sc as plsc

N = 10000
F = 128
FH = F // 2
E = 320000
ER = E // 128
BLK = 1000

NC = 2
NS = 16
ROWS_T = 157
NP = 78
DEG_N = 10240
ORPT = DEG_N // NS


def _spmm_body(ei_hbm, w_hbm, xflat_hbm,
               out_tx, out_degp, out_dis,
               src2d, dst2d, w2d, degloc, degtmp, rows,
               tx1_sh, gsem, ssem):
    c = lax.axis_index("c")
    s = lax.axis_index("s")
    xoff = c * N
    off = 156 * s + jnp.minimum(s, 4)

    zero16f = jnp.zeros((16,), jnp.float32)
    zero16i = jnp.zeros((16,), jnp.int32)

    pltpu.sync_copy(ei_hbm.at[0, pl.ds(off, 156)], src2d.at[pl.ds(0, 156)])
    pltpu.sync_copy(ei_hbm.at[1, pl.ds(off, 156)], dst2d.at[pl.ds(0, 156)])
    pltpu.sync_copy(w_hbm.at[pl.ds(off, 156)], w2d.at[pl.ds(0, 156)])

    @pl.when(s < 4)
    def _load_extra_row():
        pltpu.sync_copy(ei_hbm.at[0, pl.ds(off + 156, 1)],
                        src2d.at[pl.ds(156, 1)])
        pltpu.sync_copy(ei_hbm.at[1, pl.ds(off + 156, 1)],
                        dst2d.at[pl.ds(156, 1)])
        pltpu.sync_copy(w_hbm.at[pl.ds(off + 156, 1)], w2d.at[pl.ds(156, 1)])

    @pl.when(s >= 4)
    def _zero_tail_row():
        for k in range(8):
            src2d[156, pl.ds(16 * k, 16)] = zero16i
            dst2d[156, pl.ds(16 * k, 16)] = zero16i
            w2d[156, pl.ds(16 * k, 16)] = zero16f

    def zr(i, _):
        degloc[pl.ds(16 * i, 16)] = zero16f
        return 0
    lax.fori_loop(0, DEG_N // 16, zr, 0)

    def zb(i, _):
        rows[0, lax.div(i, 4), pl.ds(16 * lax.rem(i, 4), 16)] = zero16f
        return 0
    lax.fori_loop(0, 512, zb, 0)

    def zc(k, _):
        pltpu.sync_copy(rows.at[0], tx1_sh.at[pl.ds(ORPT * s + 128 * k, 128)])
        return 0
    lax.fori_loop(0, 5, zc, 0)

    def ar(r, _):
        for j in range(8):
            idx = src2d[r, pl.ds(16 * j, 16)]
            wv = w2d[r, pl.ds(16 * j, 16)]
            plsc.addupdate_scatter(degloc, [idx], wv)
            src2d[r, pl.ds(16 * j, 16)] = idx + xoff
        return 0
    lax.fori_loop(0, ROWS_T, ar, 0)

    pltpu.sync_copy(degloc, out_degp.at[c, s])
    plsc.subcore_barrier()

    for rnd in range(4):
        def ldr(t, _, rnd=rnd):
            pltpu.sync_copy(out_degp.at[c, 4 * rnd + t, pl.ds(ORPT * s, ORPT)],
                            degtmp.at[t])
            return 0
        lax.fori_loop(0, 4, ldr, 0)

        def mr_(i, _, rnd=rnd):
            acc = degtmp[0, pl.ds(16 * i, 16)]
            for t in range(1, 4):
                acc = acc + degtmp[t, pl.ds(16 * i, 16)]
            if rnd > 0:
                acc = acc + degloc[pl.ds(ORPT * s + 16 * i, 16)]
            if rnd == 3:
                iv = plsc.bitcast(acc, jnp.int32)
                y = plsc.bitcast(0x5F3759DF - lax.shift_right_logical(iv, 1),
                                 jnp.float32)
                for _it in range(3):
                    y = y * (1.5 - ((0.5 * acc) * y) * y)
                acc = jnp.where(acc > 0.0, y, 0.0)
            degloc[pl.ds(ORPT * s + 16 * i, 16)] = acc
            return 0
        lax.fori_loop(0, ORPT // 16, mr_, 0)

    pltpu.sync_copy(degloc.at[pl.ds(ORPT * s, ORPT)],
                    out_dis.at[c, pl.ds(ORPT * s, ORPT)])
    plsc.subcore_barrier()
    pltpu.sync_copy(out_dis.at[c], degloc)

    def nr(r, _):
        for j in range(8):
            si = src2d[r, pl.ds(16 * j, 16)]
            di = dst2d[r, pl.ds(16 * j, 16)]
            wv = w2d[r, pl.ds(16 * j, 16)]
            dsv = plsc.load_gather(degloc, [si - xoff])
            ddv = plsc.load_gather(degloc, [di])
            w2d[r, pl.ds(16 * j, 16)] = (-wv) * dsv * ddv
        return 0
    lax.fori_loop(0, ROWS_T, nr, 0)

    def scale_row(buf, r):
        def er(g, _):
            nv = w2d[r, pl.ds(16 * g, 16)]
            for e in range(16):
                nb = jnp.full((16,), nv[e], jnp.float32)
                for k in range(4):
                    rows[buf, 16 * g + e, pl.ds(16 * k, 16)] = (
                        rows[buf, 16 * g + e, pl.ds(16 * k, 16)] * nb)
            return 0
        lax.fori_loop(0, 8, er, 0, unroll=8)

    pltpu.async_copy(xflat_hbm.at[src2d.at[0]], rows.at[0], gsem.at[0])

    def cr(r, _):
        cur = lax.rem(r, 2)
        nxt = 1 - cur

        @pl.when(r >= 1)
        def _wait_prev_scatter():
            pltpu.make_async_copy(
                rows.at[nxt], tx1_sh.at[dst2d.at[r - 1]],
                ssem.at[nxt]).wait()

        @pl.when(r + 1 < ROWS_T)
        def _prefetch_next():
            pltpu.async_copy(xflat_hbm.at[src2d.at[r + 1]],
                             rows.at[nxt], gsem.at[nxt])

        pltpu.make_async_copy(xflat_hbm.at[src2d.at[r]],
                              rows.at[cur], gsem.at[cur]).wait()

        scale_row(cur, r)
        pltpu.async_copy(rows.at[cur], tx1_sh.at[dst2d.at[r]],
                         ssem.at[cur], add=True)
        return 0

    lax.fori_loop(0, ROWS_T, cr, 0)
    pltpu.make_async_copy(rows.at[(ROWS_T - 1) % 2],
                          tx1_sh.at[dst2d.at[ROWS_T - 1]],
                          ssem.at[(ROWS_T - 1) % 2]).wait()

    plsc.subcore_barrier()

    pltpu.sync_copy(tx1_sh.at[pl.ds(ORPT * s, ORPT)],
                    out_tx.at[c, pl.ds(ORPT * s, ORPT)])


@jax.jit
def _spmm_sc(ei3, w2, xflat):
    mesh = plsc.VectorSubcoreMesh(core_axis_name="c", subcore_axis_name="s",
                                  num_cores=NC, num_subcores=NS)
    return pl.kernel(
        _spmm_body,
        out_type=(
            jax.ShapeDtypeStruct((NC, DEG_N, FH), jnp.float32),
            jax.ShapeDtypeStruct((NC, NS, DEG_N), jnp.float32),
            jax.ShapeDtypeStruct((NC, DEG_N), jnp.float32),
        ),
        mesh=mesh,
        compiler_params=pltpu.CompilerParams(needs_layout_passes=False,
                                             use_tc_tiling_on_sc=False),
        scratch_types=[
            pltpu.VMEM((ROWS_T, 128), jnp.int32),
            pltpu.VMEM((ROWS_T, 128), jnp.int32),
            pltpu.VMEM((ROWS_T, 128), jnp.float32),
            pltpu.VMEM((DEG_N,), jnp.float32),
            pltpu.VMEM((4, ORPT), jnp.float32),
            pltpu.VMEM((2, 128, FH), jnp.float32),
            pltpu.VMEM_SHARED((DEG_N, FH), jnp.float32),
            pltpu.SemaphoreType.DMA((2,)),
            pltpu.SemaphoreType.DMA((2,)),
        ],
    )(ei3, w2, xflat)


def _dense_body(x_ref, t0_ref, t1_ref, wz0_ref, wz1a_ref, wz1b_ref,
                wh0_ref, wh1a_ref, wh1b_ref,
                bz_ref, bh_ref, lw_ref, lb_ref, out_ref):
    xb = x_ref[...]
    t0 = t0_ref[0]
    t1 = t1_ref[0]
    z = jax.nn.sigmoid(xb @ wz0_ref[...] + t0 @ wz1a_ref[...]
                       + t1 @ wz1b_ref[...] + bz_ref[...])
    ht = jnp.tanh(xb @ wh0_ref[...] + t0 @ wh1a_ref[...]
                  + t1 @ wh1b_ref[...] + bh_ref[...])
    h = jax.nn.relu((1.0 - z) * ht)
    out_ref[...] = h @ lw_ref[...] + lb_ref[...]


def _dense_block(x, parts, wz0, wz1, wh0, wh1, bz, bh, lw, lb):
    row_spec = pl.BlockSpec((BLK, F), lambda i: (i, 0))
    part0 = pl.BlockSpec((1, BLK, FH), lambda i: (0, i, 0))
    part1 = pl.BlockSpec((1, BLK, FH), lambda i: (1, i, 0))
    full = lambda shape: pl.BlockSpec(shape, lambda i: (0, 0))
    return pl.pallas_call(
        _dense_body,
        grid=(N // BLK,),
        in_specs=[row_spec, part0, part1,
                  full((F, F)), full((FH, F)), full((FH, F)),
                  full((F, F)), full((FH, F)), full((FH, F)),
                  full((1, F)), full((1, F)), full((F, 1)), full((1, 1))],
        out_specs=pl.BlockSpec((BLK, 1), lambda i: (i, 0)),
        out_shape=jax.ShapeDtypeStruct((N, 1), jnp.float32),
    )(x, parts, parts, wz0, wz1[:FH], wz1[FH:], wh0, wh1[:FH], wh1[FH:],
      bz, bh, lw, lb)


def kernel(x, edge_index, edge_weight,
           xz_W0, xz_W1, xz_b, hz_W0, hz_W1, hz_b,
           xr_W0, xr_W1, xr_b, hr_W0, hr_W1, hr_b,
           xh_W0, xh_W1, xh_b, hh_W0, hh_W1, hh_b,
           lin_W, lin_b):
    ei3 = edge_index.reshape(2, ER, 128)
    w2 = edge_weight.reshape(ER, 128)
    xflat = jnp.concatenate([x[:, :FH], x[:, FH:]], axis=0)

    parts, _, _ = _spmm_sc(ei3, w2, xflat)

    bz = (xz_b + hz_b).reshape(1, F)
    bh = (xh_b + hh_b).reshape(1, F)
    return _dense_block(x, parts, xz_W0, xz_W1, xh_W0, xh_W1,
                        bz, bh, lin_W, lin_b.reshape(1, 1))
